# Initial kernel scaffold; baseline (speedup 1.0000x reference)
import jax
import jax.numpy as jnp
from jax.experimental import pallas as pl


def kernel(X, edge_index_0, edge_index_1, clusters_0, L0W1, L0b1, L0W2, L0b2, L1W1, L1b1, L1W2, L1b2, R0W1, R0b1, R0W2, R0b2, R1W1, R1b1, R1W2, R1b2, Lg0, Lg1, Rg0, Rg1, Lbn0, Lbn1, Rbn0, Rbn1, M0W, M0b, M1W, M1b, M2W, M2b):
    raise NotImplementedError("write your pallas kernel here")


# speedup vs baseline: 1.4543x; 1.4543x over previous
import functools
import jax
import jax.numpy as jnp
from jax import lax
from jax.experimental import pallas as pl
from jax.experimental.pallas import tpu as pltpu
from jax.experimental.pallas import tpu_sc as plsc

NC, NS, L = 2, 16, 16
NW = NC * NS
D = 64
CR = 2000
CG = 512
EPS = 1e-5
PD = lax.Precision.DEFAULT
PH = lax.Precision.HIGHEST

_SC_PARAMS = dict(
    compiler_params=pltpu.CompilerParams(
        needs_layout_passes=False, use_tc_tiling_on_sc=False),
)


def _mesh():
    return plsc.VectorSubcoreMesh(core_axis_name="c", subcore_axis_name="s")


def _wid():
    return lax.axis_index("s") * NC + lax.axis_index("c")


def _sread(vec_ref, i):
    return vec_ref[pl.ds(i, 16)][0]


def _cdiv(a, b):
    return (a + b - 1) // b


def _route(ei, E, P, N):
    EF = E + 4 * CG
    nch = E // CR
    assert nch * CR == E

    @functools.partial(
        pl.kernel, mesh=_mesh(), **_SC_PARAMS,
        out_type=[jax.ShapeDtypeStruct((NW, 16), jnp.int32),
                  jax.ShapeDtypeStruct((NW, EF), jnp.int32),
                  jax.ShapeDtypeStruct((NW, EF), jnp.int32)],
        scratch_types=[pltpu.VMEM((CR,), jnp.int32),
                       pltpu.VMEM((CR,), jnp.int32),
                       pltpu.VMEM((CR + 32,), jnp.int32),
                       pltpu.VMEM((CR + 32,), jnp.int32),
                       pltpu.VMEM((CG,), jnp.int32),
                       pltpu.VMEM((16,), jnp.int32)],
    )
    def k(ei_hbm, cnt_out, dloc_out, src_out, dstv, srcv, bufD, bufS, zbuf,
          tmp):
        t = _wid()
        lo = t * P

        def zs(i, _):
            zbuf[pl.ds(i * 16, 16)] = jnp.zeros((16,), jnp.int32)
            return 0
        lax.fori_loop(0, CG // 16, zs, 0, unroll=False)

        def chunk(c, carry):
            w, off = carry
            pltpu.sync_copy(ei_hbm.at[1, pl.ds(c * CR, CR)], dstv)
            pltpu.sync_copy(ei_hbm.at[0, pl.ds(c * CR, CR)], srcv)

            def step(i, w):
                v = dstv[pl.ds(i * 16, 16)]
                sv = srcv[pl.ds(i * 16, 16)]
                dl = v - lo
                m = (dl >= 0) & (dl < P)
                mi = jnp.where(m, jnp.int32(1), jnp.int32(0))
                pos = w + plsc.cumsum(mi) - 1
                plsc.store_scatter(bufD, [pos], dl, mask=m)
                plsc.store_scatter(bufS, [pos], sv, mask=m)
                return pos[15] + 1

            w = lax.fori_loop(0, CR // 16, step, w, unroll=False)
            w16 = (w // 16) * 16
            offm = pl.multiple_of(off, 16)
            pltpu.sync_copy(bufD.at[pl.ds(0, CR)],
                            dloc_out.at[t, pl.ds(offm, CR)])
            pltpu.sync_copy(bufS.at[pl.ds(0, CR)],
                            src_out.at[t, pl.ds(offm, CR)])
            bufD[pl.ds(0, 16)] = bufD[pl.ds(w16, 16)]
            bufS[pl.ds(0, 16)] = bufS[pl.ds(w16, 16)]
            return w - w16, off + w16

        w, off = lax.fori_loop(0, nch, chunk, (jnp.int32(0), jnp.int32(0)),
                               unroll=False)
        lanem = lax.iota(jnp.int32, 16) < w
        bufD[pl.ds(0, 16)] = jnp.where(lanem, bufD[pl.ds(0, 16)], 0)
        bufS[pl.ds(0, 16)] = jnp.where(lanem, bufS[pl.ds(0, 16)], 0)
        offm = pl.multiple_of(off, 16)
        pltpu.sync_copy(bufD.at[pl.ds(0, 16)],
                        dloc_out.at[t, pl.ds(offm, 16)])
        pltpu.sync_copy(bufS.at[pl.ds(0, 16)],
                        src_out.at[t, pl.ds(offm, 16)])
        off16 = pl.multiple_of(off + 16, 16)
        pltpu.sync_copy(zbuf, dloc_out.at[t, pl.ds(off16, CG)])
        pltpu.sync_copy(zbuf, src_out.at[t, pl.ds(off16, CG)])
        tmp[...] = jnp.full((16,), off + w, jnp.int32)
        pltpu.sync_copy(tmp, cnt_out.at[t])

    return k(ei)


def _gather(T, dloc_priv, src_priv, cnt, off, EP, P, N, W):
    CGW = CG if W == 64 else CG // 2

    @functools.partial(
        pl.kernel, mesh=_mesh(), **_SC_PARAMS,
        out_type=[jax.ShapeDtypeStruct((EP, W), jnp.float32),
                  jax.ShapeDtypeStruct((EP, W), jnp.float32)],
        scratch_types=[pltpu.VMEM((CGW,), jnp.int32),
                       pltpu.VMEM((CGW,), jnp.int32),
                       pltpu.VMEM((CGW, W), jnp.float32),
                       pltpu.VMEM((CGW, W), jnp.float32),
                       pltpu.VMEM((96,), jnp.int32),
                       pltpu.SemaphoreType.DMA,
                       pltpu.SemaphoreType.DMA],
    )
    def k(T_hbm, dl_hbm, sp_hbm, cnt_hbm, off_hbm, Td, Ts,
          idxD, idxS, bufA, bufB, sc, semA, semB):
        t = _wid()
        lo = t * P
        pltpu.sync_copy(off_hbm.at[0], sc.at[pl.ds(0, 80)])
        pltpu.sync_copy(cnt_hbm.at[t], sc.at[pl.ds(80, 16)])
        goff = _sread(sc, t)
        my_cnt = _sread(sc, 80)
        nch = _cdiv(my_cnt + 1, CGW)

        def chunk(c, _):
            base = c * CGW
            pltpu.sync_copy(dl_hbm.at[t, pl.ds(base, CGW)], idxD)
            pltpu.sync_copy(sp_hbm.at[t, pl.ds(base, CGW)], idxS)

            def clamp(i, _):
                dv = idxD[pl.ds(i * 16, 16)] + lo
                idxD[pl.ds(i * 16, 16)] = jnp.minimum(
                    jnp.maximum(dv, 0), N - 1)
                sv = idxS[pl.ds(i * 16, 16)]
                idxS[pl.ds(i * 16, 16)] = jnp.minimum(
                    jnp.maximum(sv, 0), N - 1)
                return 0
            lax.fori_loop(0, CGW // 16, clamp, 0, unroll=False)
            cpA = pltpu.async_copy(T_hbm.at[idxD], bufA, semA)
            cpB = pltpu.async_copy(T_hbm.at[idxS], bufB, semB)
            cpA.wait()
            cpB.wait()
            gb = pl.multiple_of(goff + base, CGW)
            pltpu.sync_copy(bufA, Td.at[pl.ds(gb, CGW)])
            pltpu.sync_copy(bufB, Ts.at[pl.ds(gb, CGW)])
            return 0

        lax.fori_loop(0, nch, chunk, 0, unroll=False)

    return k(T, dloc_priv, src_priv, cnt, off)


def _scatter(m, dloc_priv, cnt, off, P, N):
    @functools.partial(
        pl.kernel, mesh=_mesh(), **_SC_PARAMS,
        out_type=[jax.ShapeDtypeStruct((NW, P, D), jnp.float32),
                  jax.ShapeDtypeStruct((NW, 128), jnp.float32)],
        scratch_types=[pltpu.VMEM((P, D), jnp.float32),
                       pltpu.VMEM((CG, D), jnp.float32),
                       pltpu.VMEM((CG,), jnp.int32),
                       pltpu.VMEM((96,), jnp.int32),
                       pltpu.VMEM((128,), jnp.float32)],
    )
    def k(m_hbm, dl_hbm, cnt_hbm, off_hbm, xr_out, st_out, tab, mv, dv, sc,
          sv):
        t = _wid()
        pltpu.sync_copy(off_hbm.at[0], sc.at[pl.ds(0, 80)])
        pltpu.sync_copy(cnt_hbm.at[t], sc.at[pl.ds(80, 16)])
        goff = _sread(sc, t)
        my_cnt = _sread(sc, 80)

        def zs(i, _):
            for kk in range(D // 16):
                tab[i, pl.ds(kk * 16, 16)] = jnp.zeros((16,), jnp.float32)
            return 0
        lax.fori_loop(0, P, zs, 0, unroll=False)

        nch = _cdiv(my_cnt, CG)

        def chunk(c, _):
            base = c * CG
            gb = pl.multiple_of(goff + base, 512)
            pltpu.sync_copy(m_hbm.at[pl.ds(gb, CG)], mv)
            pltpu.sync_copy(dl_hbm.at[t, pl.ds(base, CG)], dv)
            nb = jnp.minimum(my_cnt - base, CG)

            def es(i, _):
                d = _sread(dv, i)
                for kk in range(D // 16):
                    sl = pl.ds(kk * 16, 16)
                    tab[d, sl] = jnp.maximum(tab[d, sl], mv[i, sl])
                return 0
            lax.fori_loop(0, nb, es, 0, unroll=False)
            return 0

        lax.fori_loop(0, nch, chunk, 0, unroll=False)
        pltpu.sync_copy(tab, xr_out.at[t])

        nrows = jnp.minimum(P, N - t * P)
        zero = jnp.zeros((16,), jnp.float32)

        def rs(i, acc):
            a0, a1, a2, a3, q0, q1, q2, q3 = acc
            r0 = tab[i, pl.ds(0, 16)]
            r1 = tab[i, pl.ds(16, 16)]
            r2 = tab[i, pl.ds(32, 16)]
            r3 = tab[i, pl.ds(48, 16)]
            return (a0 + r0, a1 + r1, a2 + r2, a3 + r3,
                    q0 + r0 * r0, q1 + r1 * r1, q2 + r2 * r2, q3 + r3 * r3)

        acc = lax.fori_loop(0, nrows, rs, (zero,) * 8, unroll=False)
        for kk in range(4):
            sv[pl.ds(kk * 16, 16)] = acc[kk]
            sv[pl.ds(64 + kk * 16, 16)] = acc[kk + 4]
        pltpu.sync_copy(sv, st_out.at[t])

    return k(m, dloc_priv, cnt, off)


def _pool(x0p, clp, N0P, N1P):
    RPT = N0P // NW

    @functools.partial(
        pl.kernel, mesh=_mesh(), **_SC_PARAMS,
        out_type=[jax.ShapeDtypeStruct((NC, N1P, D), jnp.float32),
                  jax.ShapeDtypeStruct((NC, N1P, 16), jnp.float32)],
        scratch_types=[pltpu.VMEM((RPT, D), jnp.float32),
                       pltpu.VMEM((RPT, 16), jnp.float32),
                       pltpu.VMEM((RPT,), jnp.int32),
                       pltpu.VMEM((16, D), jnp.float32),
                       pltpu.VMEM_SHARED((N1P, D), jnp.float32),
                       pltpu.VMEM_SHARED((N1P, 16), jnp.float32)],
    )
    def k(x_hbm, cl_hbm, sum_out, cntp_out, rows, ones, idx, zb, shS, shC):
        cid = lax.axis_index("c")
        sid = lax.axis_index("s")
        t = sid * NC + cid

        @pl.when(sid == 0)
        def _():
            def zz(i, _):
                for kk in range(D // 16):
                    zb[i, pl.ds(kk * 16, 16)] = jnp.zeros((16,), jnp.float32)
                return 0
            lax.fori_loop(0, 16, zz, 0, unroll=False)

            def zfill(i, _):
                pltpu.sync_copy(zb, shS.at[pl.ds(i * 16, 16)])
                pltpu.sync_copy(zb.at[:, pl.ds(0, 16)],
                                shC.at[pl.ds(i * 16, 16)])
                return 0
            lax.fori_loop(0, N1P // 16, zfill, 0, unroll=False)

        def os_(i, _):
            ones[i, pl.ds(0, 16)] = jnp.full((16,), 1.0, jnp.float32)
            return 0
        lax.fori_loop(0, RPT, os_, 0, unroll=False)

        plsc.subcore_barrier()
        base = t * RPT
        pltpu.sync_copy(x_hbm.at[pl.ds(base, RPT)], rows)
        pltpu.sync_copy(cl_hbm.at[pl.ds(base, RPT)], idx)
        pltpu.sync_copy(rows, shS.at[idx], add=True)
        pltpu.sync_copy(ones, shC.at[idx], add=True)
        plsc.subcore_barrier()

        @pl.when(sid == 0)
        def _():
            pltpu.sync_copy(shS, sum_out.at[cid])
            pltpu.sync_copy(shC, cntp_out.at[cid])

    return k(x0p, clp)


def _unpool(y, clp, N0P, N1):
    RPT = N0P // NW

    @functools.partial(
        pl.kernel, mesh=_mesh(), **_SC_PARAMS,
        out_type=jax.ShapeDtypeStruct((N0P, D), jnp.float32),
        scratch_types=[pltpu.VMEM((RPT,), jnp.int32),
                       pltpu.VMEM((RPT, D), jnp.float32),
                       pltpu.SemaphoreType.DMA],
    )
    def k(y_hbm, cl_hbm, g_out, idx, bA, sA):
        t = _wid()
        base = t * RPT
        pltpu.sync_copy(cl_hbm.at[pl.ds(base, RPT)], idx)

        def clamp(i, _):
            v = idx[pl.ds(i * 16, 16)]
            idx[pl.ds(i * 16, 16)] = jnp.minimum(jnp.maximum(v, 0), N1 - 1)
            return 0
        lax.fori_loop(0, RPT // 16, clamp, 0, unroll=False)
        cpA = pltpu.async_copy(y_hbm.at[idx], bA, sA)
        cpA.wait()
        pltpu.sync_copy(bA, g_out.at[pl.ds(base, RPT)])

    return k(y, clp)


def _tc_call(body, out_shape, *args, grid=None, in_specs=None,
             out_specs=None):
    kw = {}
    if grid is not None:
        kw = dict(grid=grid, in_specs=in_specs, out_specs=out_specs)
    return pl.pallas_call(body, out_shape=out_shape, **kw)(*args)


def _offsets(cnt0, cnt1):
    def body(c0_ref, c1_ref, o0_ref, o1_ref):
        ii = lax.broadcasted_iota(jnp.int32, (80, 32), 0)
        jj = lax.broadcasted_iota(jnp.int32, (80, 32), 1)
        tri = jnp.where(jj < ii, 1.0, 0.0)
        for c_ref, o_ref in ((c0_ref, o0_ref), (c1_ref, o1_ref)):
            c = c_ref[...][:, 0]
            al = (((c + 512) // 512) * 512).astype(jnp.float32)
            off = jnp.dot(tri, al[:, None],
                          preferred_element_type=jnp.float32, precision=PH)
            o_ref[...] = off.astype(jnp.int32).reshape(1, 80)

    return _tc_call(
        body,
        [jax.ShapeDtypeStruct((1, 80), jnp.int32),
         jax.ShapeDtypeStruct((1, 80), jnp.int32)],
        cnt0, cnt1)


def _edge_mlp(Td, Ts, W1a, W1b, b1, W2, b2):
    EP, F = Td.shape
    BLK = 2048
    assert EP % BLK == 0

    def body(td_ref, ts_ref, wa_ref, wb_ref, b1_ref, w2_ref, b2_ref, m_ref):
        xi = td_ref[...]
        dj = ts_ref[...] - xi
        h = jnp.maximum(
            jnp.dot(xi, wa_ref[...],
                    preferred_element_type=jnp.float32, precision=PD)
            + jnp.dot(dj, wb_ref[...],
                      preferred_element_type=jnp.float32, precision=PD)
            + b1_ref[...], 0.0)
        m_ref[...] = jnp.dot(h, w2_ref[...],
                             preferred_element_type=jnp.float32,
                             precision=PD) + b2_ref[...]

    return _tc_call(
        body, jax.ShapeDtypeStruct((EP, D), jnp.float32),
        Td, Ts, W1a, W1b, b1.reshape(1, D), W2, b2.reshape(1, D),
        grid=(EP // BLK,),
        in_specs=[pl.BlockSpec((BLK, F), lambda i: (i, 0)),
                  pl.BlockSpec((BLK, F), lambda i: (i, 0)),
                  pl.BlockSpec((F, D), lambda i: (0, 0)),
                  pl.BlockSpec((F, D), lambda i: (0, 0)),
                  pl.BlockSpec((1, D), lambda i: (0, 0)),
                  pl.BlockSpec((D, D), lambda i: (0, 0)),
                  pl.BlockSpec((1, D), lambda i: (0, 0))],
        out_specs=pl.BlockSpec((BLK, D), lambda i: (i, 0)))


def _bn_scale_shift(st, g, b, n):
    s = jnp.sum(st[:, :D], axis=0, keepdims=True)
    q = jnp.sum(st[:, D:], axis=0, keepdims=True)
    mu = s / n
    var = q / n - mu * mu
    sc = g.reshape(1, D) * lax.rsqrt(var + EPS)
    sh = b.reshape(1, D) - mu * sc
    return sc, sh


def _bn_apply(xr, st, g, b, n):
    def body(x_ref, st_ref, g_ref, b_ref, o_ref):
        sc, sh = _bn_scale_shift(st_ref[...], g_ref[...], b_ref[...], n)
        o_ref[...] = x_ref[...] * sc + sh

    return _tc_call(
        body, jax.ShapeDtypeStruct(xr.shape, jnp.float32),
        xr, st, g.reshape(1, D), b.reshape(1, D))


def _pool_div(sums, cnts):
    def body(s_ref, c_ref, o_ref):
        s = s_ref[0] + s_ref[1]
        c = c_ref[0][:, 0:1] + c_ref[1][:, 0:1]
        o_ref[...] = s / jnp.maximum(c, 1.0)

    N1 = sums.shape[1]
    return _tc_call(
        body, jax.ShapeDtypeStruct((N1, D), jnp.float32), sums, cnts)


def _final_mlp(xr, st, g, b, n, M0W, M0b, M1W, M1b, M2W, M2b):
    N = xr.shape[0]
    BLK = 2000

    def body(x_ref, st_ref, g_ref, b_ref, w0, b0, w1, b1_, w2, b2_, o_ref):
        sc, sh = _bn_scale_shift(st_ref[...], g_ref[...], b_ref[...], n)
        x = x_ref[...] * sc + sh
        h = jnp.maximum(jnp.dot(x, w0[...],
                                preferred_element_type=jnp.float32,
                                precision=PD)
                        + b0[...], 0.0)
        h = jnp.maximum(jnp.dot(h, w1[...],
                                preferred_element_type=jnp.float32,
                                precision=PD)
                        + b1_[...], 0.0)
        o_ref[...] = jnp.dot(h, w2[...],
                             preferred_element_type=jnp.float32,
                             precision=PD) + b2_[...]

    return _tc_call(
        body, jax.ShapeDtypeStruct((N, D), jnp.float32),
        xr, st, g.reshape(1, D), b.reshape(1, D),
        M0W, M0b.reshape(1, 128), M1W, M1b.reshape(1, 128),
        M2W, M2b.reshape(1, D),
        grid=(N // BLK,),
        in_specs=[pl.BlockSpec((BLK, D), lambda i: (i, 0)),
                  pl.BlockSpec((NW, 128), lambda i: (0, 0)),
                  pl.BlockSpec((1, D), lambda i: (0, 0)),
                  pl.BlockSpec((1, D), lambda i: (0, 0)),
                  pl.BlockSpec((D, 128), lambda i: (0, 0)),
                  pl.BlockSpec((1, 128), lambda i: (0, 0)),
                  pl.BlockSpec((128, 128), lambda i: (0, 0)),
                  pl.BlockSpec((1, 128), lambda i: (0, 0)),
                  pl.BlockSpec((128, D), lambda i: (0, 0)),
                  pl.BlockSpec((1, D), lambda i: (0, 0))],
        out_specs=pl.BlockSpec((BLK, D), lambda i: (i, 0)))


def _conv(T, route, W1a, W1b, b1, W2, b2, EP, P, N):
    cnt, dloc_priv, src_priv, off = route
    W = T.shape[1]
    Td, Ts = _gather(T, dloc_priv, src_priv, cnt, off, EP, P, N, W)
    m = _edge_mlp(Td, Ts, W1a, W1b, b1, W2, b2)
    xr, st = _scatter(m, dloc_priv, cnt, off, P, N)
    return xr, st


def kernel(X, edge_index_0, edge_index_1, clusters_0,
           L0W1, L0b1, L0W2, L0b2, L1W1, L1b1, L1W2, L1b2,
           R0W1, R0b1, R0W2, R0b2, R1W1, R1b1, R1W2, R1b2,
           Lg0, Lg1, Rg0, Rg1, Lbn0, Lbn1, Rbn0, Rbn1,
           M0W, M0b, M1W, M1b, M2W, M2b):
    N0, N1 = X.shape[0], 2000
    E0 = edge_index_0.shape[1]
    E1 = edge_index_1.shape[1]
    P0 = _cdiv(N0, NW)
    P1 = _cdiv(N1, NW)
    N0P = P0 * NW
    EP0 = ((E0 + NW * CG + 2047) // 2048) * 2048
    EP1 = ((E1 + NW * CG + 2047) // 2048) * 2048

    cnt0, dl0, sp0 = _route(edge_index_0, E0, P0, N0)
    cnt1, dl1, sp1 = _route(edge_index_1, E1, P1, N1)
    off0, off1 = _offsets(cnt0, cnt1)
    r0 = (cnt0, dl0, sp0, off0)
    r1 = (cnt1, dl1, sp1, off1)

    xr0, st0 = _conv(X, r0, L0W1[:D], L0W1[D:], L0b1, L0W2, L0b2,
                     EP0, P0, N0)
    xr0 = xr0.reshape(N0P, D)[:N0]
    x0 = _bn_apply(xr0, st0, Lg0, Lbn0, float(N0))

    N0G = 10240
    x0p = jnp.zeros((N0G, D), jnp.float32).at[:N0].set(x0)
    clp = jnp.full((N0G,), 2008, jnp.int32).at[:N0].set(clusters_0)
    N1P = 2016
    sums, cnts = _pool(x0p, clp, N0G, N1P)
    xp = _pool_div(sums[:, :N1], cnts[:, :N1])

    xr1, st1 = _conv(xp, r1, L1W1[:D], L1W1[D:], L1b1, L1W2, L1b2,
                     EP1, P1, N1)
    xr1 = xr1.reshape(P1 * NW, D)[:N1]
    x1 = _bn_apply(xr1, st1, Lg1, Lbn1, float(N1))

    xT1 = jnp.concatenate([x1, x1], axis=1)
    yr, sty = _conv(xT1, r1, R1W1[:128], R1W1[128:], R1b1, R1W2, R1b2,
                    EP1, P1, N1)
    yr = yr.reshape(P1 * NW, D)[:N1]
    y = _bn_apply(yr, sty, Rg1, Rbn1, float(N1))

    yc = _unpool(y, clp, N0G, N1)
    xT0 = jnp.concatenate([x0, yc[:N0]], axis=1)
    xrF, stF = _conv(xT0, r0, R0W1[:128], R0W1[128:], R0b1, R0W2, R0b2,
                     EP0, P0, N0)
    xrF = xrF.reshape(N0P, D)[:N0]

    return _final_mlp(xrF, stF, Rg0, Rbn0, float(N0),
                      M0W, M0b, M1W, M1b, M2W, M2b)
